# baseline (speedup 1.0000x reference)
import functools

import jax
import jax.numpy as jnp
from jax.experimental import pallas as pl

_SPARSITY_FRAC = 0.05


def _scores_kernel(emb_ref, w1_ref, b1_ref, w2_ref, out_ref):
    e = emb_ref[0].astype(jnp.bfloat16)
    h = jnp.maximum(
        jnp.dot(e, w1_ref[...].astype(jnp.bfloat16),
                preferred_element_type=jnp.float32) + b1_ref[...],
        0.0,
    )
    hw = h.astype(jnp.bfloat16).astype(jnp.float32) * (
        w2_ref[...].astype(jnp.bfloat16).astype(jnp.float32))
    out_ref[0] = jnp.sum(hw, axis=1, keepdims=True)


def _topk_kernel(k, S, JBLK, P, CH, sj_ref, scol_ref, mask_ref, tidx_ref):
    jb = pl.program_id(1)
    s_j = sj_ref[0]
    j_idx = jb * JBLK + jax.lax.broadcasted_iota(jnp.int32, (1, JBLK), 1)

    def body(c, acc):
        s_i = scol_ref[0, pl.ds(c * CH, CH), :]
        i_idx = c * CH + jax.lax.broadcasted_iota(jnp.int32, (CH, 1), 0)
        ahead = (s_i > s_j) | ((s_i == s_j) & (i_idx < j_idx))
        return acc + jnp.sum(ahead.astype(jnp.int32), axis=0, keepdims=True)

    rank = jax.lax.fori_loop(0, S // CH, body, jnp.zeros((1, JBLK), jnp.int32))
    mask_ref[0] = jnp.where(rank < k, 1.0 / k, 0.0)

    p = jax.lax.broadcasted_iota(jnp.int32, (P, 1), 0)
    contrib = jnp.sum(
        jnp.where(rank == p, jnp.broadcast_to(j_idx, (P, JBLK)), 0),
        axis=1,
        keepdims=True,
    )

    @pl.when(jb == 0)
    def _():
        tidx_ref[...] = jnp.zeros_like(tidx_ref)

    tidx_ref[...] += contrib[None]


def _bcast_kernel(mask_ref, out_ref):
    out_ref[...] = jnp.broadcast_to(mask_ref[...], out_ref.shape)


def kernel(embeddings, W1, b1, W2, b2):
    B, S, D = embeddings.shape
    F = W1.shape[1]
    k = max(1, int(S * _SPARSITY_FRAC))

    MB = 512
    scores_col = pl.pallas_call(
        _scores_kernel,
        grid=(B, S // MB),
        in_specs=[
            pl.BlockSpec((1, MB, D), lambda b, m: (b, m, 0)),
            pl.BlockSpec((D, F), lambda b, m: (0, 0)),
            pl.BlockSpec((1, F), lambda b, m: (0, 0)),
            pl.BlockSpec((1, F), lambda b, m: (0, 0)),
        ],
        out_specs=pl.BlockSpec((1, MB, 1), lambda b, m: (b, m, 0)),
        out_shape=jax.ShapeDtypeStruct((B, S, 1), jnp.float32),
    )(embeddings, W1, b1.reshape(1, F), W2.reshape(1, F))
    scores_row = scores_col.reshape(B, 1, S)

    JBLK = 512
    P = 256
    CH = 512
    mask_row, tidx = pl.pallas_call(
        functools.partial(_topk_kernel, k, S, JBLK, P, CH),
        grid=(B, S // JBLK),
        in_specs=[
            pl.BlockSpec((1, 1, JBLK), lambda b, j: (b, 0, j)),
            pl.BlockSpec((1, S, 1), lambda b, j: (b, 0, 0)),
        ],
        out_specs=[
            pl.BlockSpec((1, 1, JBLK), lambda b, j: (b, 0, j)),
            pl.BlockSpec((1, P, 1), lambda b, j: (b, 0, 0)),
        ],
        out_shape=[
            jax.ShapeDtypeStruct((B, 1, S), jnp.float32),
            jax.ShapeDtypeStruct((B, P, 1), jnp.int32),
        ],
    )(scores_row, scores_col)

    R = 256
    attn = pl.pallas_call(
        _bcast_kernel,
        grid=(B, S // R),
        in_specs=[pl.BlockSpec((1, 1, S), lambda b, r: (b, 0, 0))],
        out_specs=pl.BlockSpec((1, R, S), lambda b, r: (b, r, 0)),
        out_shape=jax.ShapeDtypeStruct((B, S, S), jnp.float32),
    )(mask_row)

    top_indices = tidx[:, :k, 0]
    return attn, top_indices


# speedup vs baseline: 1.3363x; 1.3363x over previous
import functools

import jax
import jax.numpy as jnp
from jax.experimental import pallas as pl

_SPARSITY_FRAC = 0.05


def _scores_kernel(emb_ref, w1_ref, b1_ref, w2_ref, out_ref):
    e = emb_ref[0].astype(jnp.bfloat16)
    h = jnp.maximum(
        jnp.dot(e, w1_ref[...].astype(jnp.bfloat16),
                preferred_element_type=jnp.float32) + b1_ref[...],
        0.0,
    )
    hw = h.astype(jnp.bfloat16).astype(jnp.float32) * (
        w2_ref[...].astype(jnp.bfloat16).astype(jnp.float32))
    out_ref[0] = jnp.sum(hw, axis=1, keepdims=True)


def _excl_prefix(x, S):
    ii = jax.lax.broadcasted_iota(jnp.int32, (128, 128), 0)
    jj = jax.lax.broadcasted_iota(jnp.int32, (128, 128), 1)
    lt = (ii < jj).astype(jnp.float32)
    chunks = []
    base = jnp.zeros((1, 1), jnp.float32)
    for c in range(S // 128):
        ch = x[:, c * 128:(c + 1) * 128]
        pw = jax.lax.dot_general(ch, lt, (((1,), (0,)), ((), ())),
                                 preferred_element_type=jnp.float32,
                                 precision=jax.lax.Precision.HIGHEST)
        chunks.append(pw + base)
        base = base + jnp.sum(ch, axis=1, keepdims=True)
    return jnp.concatenate(chunks, axis=1)


def _topk_body(k, S, P, s):
    bits = jax.lax.bitcast_convert_type(s, jnp.int32)
    key = bits ^ (jax.lax.shift_right_arithmetic(bits, 31) & jnp.int32(0x7FFFFFFF))

    nonneg = (key >= 0).astype(jnp.int32)
    cnt0 = jnp.sum(nonneg)
    take_hi = k <= cnt0
    active = jnp.where(take_hi, nonneg, 1 - nonneg)
    kk = jnp.where(take_hi, k, k - cnt0)
    T = jnp.where(take_hi, jnp.int32(0), jnp.int32(-2147483648))

    def bit_body(bi, carry):
        active, kk, T = carry
        b = 30 - bi
        bitset = jax.lax.shift_right_arithmetic(key, b) & 1
        hi = active * bitset
        cnt = jnp.sum(hi)
        take = kk <= cnt
        active = jnp.where(take, hi, active * (1 - bitset))
        kk = jnp.where(take, kk, kk - cnt)
        T = jnp.where(take, T | jax.lax.shift_left(jnp.int32(1), b), T)
        return active, kk, T

    _, _, T = jax.lax.fori_loop(0, 31, bit_body, (active, kk, T))

    gt = key > T
    eq = key == T
    ngt = jnp.sum(gt.astype(jnp.int32))
    m = (k - ngt).astype(jnp.float32)
    tie_pref = _excl_prefix(eq.astype(jnp.float32), S)
    sel = jnp.logical_or(gt, jnp.logical_and(eq, tie_pref < m))
    mask_row = jnp.where(sel, jnp.float32(1.0 / k), jnp.float32(0.0))

    c_row = _excl_prefix(sel.astype(jnp.float32), S)
    p_col = jax.lax.broadcasted_iota(jnp.int32, (P, 1), 0).astype(jnp.float32)
    onehot = jnp.logical_and(c_row == p_col, sel)
    j_row = jax.lax.broadcasted_iota(jnp.int32, (1, S), 1).astype(jnp.float32)
    hi_row = jax.lax.shift_right_arithmetic(key, 16).astype(jnp.float32)
    lo_row = (key & jnp.int32(0xFFFF)).astype(jnp.float32)
    ohf = onehot.astype(jnp.float32)
    cand_idx = jnp.sum(ohf * j_row, axis=1, keepdims=True)
    cand_hi = jnp.sum(ohf * hi_row, axis=1, keepdims=True)
    cand_lo = jnp.sum(ohf * lo_row, axis=1, keepdims=True)

    ee = jax.lax.broadcasted_iota(jnp.int32, (P, P), 0)
    ff = jax.lax.broadcasted_iota(jnp.int32, (P, P), 1)
    eye = (ee == ff).astype(jnp.float32)
    tdims = (((0,), (0,)), ((), ()))
    hp = jax.lax.Precision.HIGHEST
    cand_idx_r = jax.lax.dot_general(cand_idx, eye, tdims,
                                     preferred_element_type=jnp.float32,
                                     precision=hp)
    cand_hi_r = jax.lax.dot_general(cand_hi, eye, tdims,
                                    preferred_element_type=jnp.float32,
                                    precision=hp)
    cand_lo_r = jax.lax.dot_general(cand_lo, eye, tdims,
                                    preferred_element_type=jnp.float32,
                                    precision=hp)

    valid_c = p_col < k
    valid_r = jax.lax.broadcasted_iota(jnp.int32, (1, P), 1) < k
    ahead = jnp.logical_or(
        cand_hi > cand_hi_r,
        jnp.logical_and(
            cand_hi == cand_hi_r,
            jnp.logical_or(
                cand_lo > cand_lo_r,
                jnp.logical_and(cand_lo == cand_lo_r, cand_idx < cand_idx_r),
            ),
        ),
    )
    ahead = jnp.logical_and(ahead, jnp.logical_and(valid_c, valid_r))
    rank_r = jnp.sum(ahead.astype(jnp.float32), axis=0, keepdims=True)
    rank_r = jnp.where(valid_r, rank_r, jnp.float32(1e9))

    hit = (rank_r == p_col).astype(jnp.float32)
    tidx = jnp.sum(hit * cand_idx_r, axis=1, keepdims=True).astype(jnp.int32)
    return mask_row, tidx


def _topk_kernel(k, S, P, srow_ref, mask_ref, tidx_ref):
    mask_row, tidx = _topk_body(k, S, P, srow_ref[0])
    mask_ref[0] = mask_row
    tidx_ref[0] = tidx


def _bcast_kernel(mask_ref, out_ref):
    out_ref[...] = jnp.broadcast_to(mask_ref[...], out_ref.shape)


def kernel(embeddings, W1, b1, W2, b2):
    B, S, D = embeddings.shape
    F = W1.shape[1]
    k = max(1, int(S * _SPARSITY_FRAC))

    MB = 512
    scores_col = pl.pallas_call(
        _scores_kernel,
        grid=(B, S // MB),
        in_specs=[
            pl.BlockSpec((1, MB, D), lambda b, m: (b, m, 0)),
            pl.BlockSpec((D, F), lambda b, m: (0, 0)),
            pl.BlockSpec((1, F), lambda b, m: (0, 0)),
            pl.BlockSpec((1, F), lambda b, m: (0, 0)),
        ],
        out_specs=pl.BlockSpec((1, MB, 1), lambda b, m: (b, m, 0)),
        out_shape=jax.ShapeDtypeStruct((B, S, 1), jnp.float32),
    )(embeddings, W1, b1.reshape(1, F), W2.reshape(1, F))
    scores_row = scores_col.reshape(B, 1, S)

    P = 256
    mask_row, tidx = pl.pallas_call(
        functools.partial(_topk_kernel, k, S, P),
        grid=(B,),
        in_specs=[pl.BlockSpec((1, 1, S), lambda b: (b, 0, 0))],
        out_specs=[
            pl.BlockSpec((1, 1, S), lambda b: (b, 0, 0)),
            pl.BlockSpec((1, P, 1), lambda b: (b, 0, 0)),
        ],
        out_shape=[
            jax.ShapeDtypeStruct((B, 1, S), jnp.float32),
            jax.ShapeDtypeStruct((B, P, 1), jnp.int32),
        ],
    )(scores_row)

    R = 256
    attn = pl.pallas_call(
        _bcast_kernel,
        grid=(B, S // R),
        in_specs=[pl.BlockSpec((1, 1, S), lambda b, r: (b, 0, 0))],
        out_specs=pl.BlockSpec((1, R, S), lambda b, r: (b, r, 0)),
        out_shape=jax.ShapeDtypeStruct((B, S, S), jnp.float32),
    )(mask_row)

    top_indices = tidx[:, :k, 0]
    return attn, top_indices
